# Initial kernel scaffold; baseline (speedup 1.0000x reference)
import jax
import jax.numpy as jnp
from jax.experimental import pallas as pl


def kernel(x, vals, rows, cols):
    raise NotImplementedError("write your pallas kernel here")


# speedup vs baseline: 99.0552x; 99.0552x over previous
import functools

import jax
import jax.numpy as jnp
from jax import lax
from jax.experimental import pallas as pl
from jax.experimental.pallas import tpu as pltpu
from jax.experimental.pallas import tpu_sc as plsc

Nc = 128
Ns = 2048
Nx = 256
Nz = 512
B = 4
K = 1
NB = B * K
NR = Nx * Nz
NCOL = Ns * Nc
NW = 32
RPW = NR // NW
CH = 2048


def _sc_spmm(rf, vals, rows, cols, bounds):
    mesh = plsc.VectorSubcoreMesh(
        core_axis_name="c", subcore_axis_name="s", num_cores=2)

    @functools.partial(
        pl.kernel,
        out_type=jax.ShapeDtypeStruct((NR * NB,), jnp.float32),
        mesh=mesh,
        scratch_types=[
            pltpu.VMEM((CH,), jnp.int32),
            pltpu.VMEM((CH,), jnp.int32),
            pltpu.VMEM((CH,), jnp.float32),
            pltpu.VMEM((CH, NB), jnp.float32),
            pltpu.VMEM((RPW * NB,), jnp.float32),
            pltpu.VMEM((NW, 16), jnp.int32),
            pltpu.SemaphoreType.DMA,
        ],
        compiler_params=pltpu.CompilerParams(use_tc_tiling_on_sc=False, needs_layout_passes=False),
    )
    def k(rf_h, vals_h, rows_h, cols_h, bounds_h, out_h,
          colv, rowv, valv, gath, acc, bsm, sem):
        cid = lax.axis_index("c")
        sid = lax.axis_index("s")
        wid = sid * 2 + cid
        lo = wid * RPW

        zeros = jnp.zeros((16,), jnp.float32)

        def zbody(i, _):
            acc[pl.ds(i * 16, 16)] = zeros
            return 0

        lax.fori_loop(0, RPW * NB // 16, zbody, 0)

        pltpu.sync_copy(bounds_h, bsm)
        bvec = bsm[wid, :]
        start = bvec[0]
        end = bvec[1]
        base0 = (start // CH) * CH
        nch = (end - base0 + CH - 1) // CH

        qdiv = lax.iota(jnp.int32, 16) // 4
        jlane = lax.iota(jnp.int32, 16) % 4

        def chunk(ci, _):
            base = base0 + ci * CH
            pltpu.sync_copy(cols_h.at[pl.ds(base, CH)], colv)
            cp = pltpu.async_copy(rf_h.at[colv], gath, sem)
            pltpu.sync_copy(vals_h.at[pl.ds(base, CH)], valv)
            pltpu.sync_copy(rows_h.at[pl.ds(base, CH)], rowv)
            cp.wait()

            def grp(g, _):
                idx4 = g * 4 + qdiv
                v = plsc.load_gather(valv, [idx4])
                r = plsc.load_gather(rowv, [idx4])
                gv = plsc.load_gather(gath, [idx4, jlane])
                rrel = r - lo
                m = plsc.bitcast(rrel, jnp.uint32) < jnp.uint32(RPW)
                oi = rrel * NB + jlane
                plsc.addupdate_scatter(acc, [oi], v * gv, mask=m)
                return 0

            lax.fori_loop(0, CH // 4, grp, 0)
            return 0

        lax.fori_loop(0, nch, chunk, 0)
        pltpu.sync_copy(acc, out_h.at[pl.ds(lo * NB, RPW * NB)])

    return k(rf, vals, rows, cols, bounds)


def kernel(x, vals, rows, cols):
    rf = jnp.transpose(x.reshape(NB, Nc, Ns), (2, 1, 0)).reshape(NCOL, NB)
    edges = jnp.arange(0, NR + 1, RPW, dtype=jnp.int32)
    b = jnp.searchsorted(rows, edges, method="scan_unrolled").astype(jnp.int32)
    bounds = jnp.zeros((NW, 16), jnp.int32)
    bounds = bounds.at[:, 0].set(b[:NW]).at[:, 1].set(b[1:])
    img = _sc_spmm(rf, vals, rows, cols, bounds).reshape(NR, NB)
    return jnp.transpose(img.T.reshape(B, K, Nx, Nz), (0, 1, 3, 2))


# speedup vs baseline: 104.3786x; 1.0537x over previous
import functools

import jax
import jax.numpy as jnp
from jax import lax
from jax.experimental import pallas as pl
from jax.experimental.pallas import tpu as pltpu
from jax.experimental.pallas import tpu_sc as plsc

Nc = 128
Ns = 2048
Nx = 256
Nz = 512
B = 4
K = 1
NB = B * K
NR = Nx * Nz
NCOL = Ns * Nc
NW = 32
RPW = NR // NW
CH = 8192


def _sc_spmm(rf, vals, rows, cols, bounds):
    mesh = plsc.VectorSubcoreMesh(
        core_axis_name="c", subcore_axis_name="s", num_cores=2)

    @functools.partial(
        pl.kernel,
        out_type=jax.ShapeDtypeStruct((NR * NB,), jnp.float32),
        mesh=mesh,
        scratch_types=[
            pltpu.VMEM((CH,), jnp.int32),
            pltpu.VMEM((CH,), jnp.int32),
            pltpu.VMEM((CH,), jnp.float32),
            pltpu.VMEM((CH, NB), jnp.float32),
            pltpu.VMEM((RPW * NB,), jnp.float32),
            pltpu.VMEM((NW, 16), jnp.int32),
            pltpu.SemaphoreType.DMA,
        ],
        compiler_params=pltpu.CompilerParams(use_tc_tiling_on_sc=False, needs_layout_passes=False),
    )
    def k(rf_h, vals_h, rows_h, cols_h, bounds_h, out_h,
          colv, rowv, valv, gath, acc, bsm, sem):
        cid = lax.axis_index("c")
        sid = lax.axis_index("s")
        wid = sid * 2 + cid
        lo = wid * RPW

        zeros = jnp.zeros((16,), jnp.float32)

        def zbody(i, _):
            acc[pl.ds(i * 16, 16)] = zeros
            return 0

        lax.fori_loop(0, RPW * NB // 16, zbody, 0)

        pltpu.sync_copy(bounds_h, bsm)
        bvec = bsm[wid, :]
        start = bvec[0]
        end = bvec[1]
        base0 = (start // CH) * CH
        nch = (end - base0 + CH - 1) // CH

        qdiv = lax.iota(jnp.int32, 16) // 4
        jlane = lax.iota(jnp.int32, 16) % 4

        def chunk(ci, _):
            base = base0 + ci * CH
            pltpu.sync_copy(cols_h.at[pl.ds(base, CH)], colv)
            cp = pltpu.async_copy(rf_h.at[colv], gath, sem)
            pltpu.sync_copy(vals_h.at[pl.ds(base, CH)], valv)
            pltpu.sync_copy(rows_h.at[pl.ds(base, CH)], rowv)
            cp.wait()

            def grp(g, _):
                idx4 = g * 4 + qdiv
                v = plsc.load_gather(valv, [idx4])
                r = plsc.load_gather(rowv, [idx4])
                gv = plsc.load_gather(gath, [idx4, jlane])
                rrel = r - lo
                m = plsc.bitcast(rrel, jnp.uint32) < jnp.uint32(RPW)
                oi = rrel * NB + jlane
                plsc.addupdate_scatter(acc, [oi], v * gv, mask=m)
                return 0

            lax.fori_loop(0, CH // 4, grp, 0)
            return 0

        lax.fori_loop(0, nch, chunk, 0)
        pltpu.sync_copy(acc, out_h.at[pl.ds(lo * NB, RPW * NB)])

    return k(rf, vals, rows, cols, bounds)


def kernel(x, vals, rows, cols):
    rf = jnp.transpose(x.reshape(NB, Nc, Ns), (2, 1, 0)).reshape(NCOL, NB)
    edges = jnp.arange(0, NR + 1, RPW, dtype=jnp.int32)
    b = jnp.searchsorted(rows, edges, method="scan_unrolled").astype(jnp.int32)
    bounds = jnp.zeros((NW, 16), jnp.int32)
    bounds = bounds.at[:, 0].set(b[:NW]).at[:, 1].set(b[1:])
    img = _sc_spmm(rf, vals, rows, cols, bounds).reshape(NR, NB)
    return jnp.transpose(img.T.reshape(B, K, Nx, Nz), (0, 1, 3, 2))
